# Initial kernel scaffold; baseline (speedup 1.0000x reference)
import jax
import jax.numpy as jnp
from jax.experimental import pallas as pl


def kernel(input, target):
    raise NotImplementedError("write your pallas kernel here")


# speedup vs baseline: 2.4686x; 2.4686x over previous
import functools

import jax
import jax.numpy as jnp
from jax.experimental import pallas as pl
from jax.experimental.pallas import tpu as pltpu

_SMOOTH = 0.1


def _ohem_kernel(x_ref, t_ref, o_ref, ps_ref, *, nblocks, keep):
    i = pl.program_id(0)
    x = x_ref[...]
    R, C = x.shape
    m = jnp.max(x, axis=1, keepdims=True)
    se = jnp.sum(jnp.exp(x - m), axis=1)
    lse = jnp.log(se) + m[:, 0]
    s = jnp.sum(x, axis=1)
    t = t_ref[0, 0, :]
    col = jax.lax.broadcasted_iota(jnp.int32, (R, C), 1)
    xt = jnp.sum(jnp.where(col == t[:, None], x, 0.0), axis=1)
    a = 1.0 - _SMOOTH - _SMOOTH / (C - 1)
    b = _SMOOTH / (C - 1)
    ps_ref[i, :] = lse - a * xt - b * s

    @pl.when(i == nblocks - 1)
    def _select():
        v = ps_ref[...]
        bits = jax.lax.bitcast_convert_type(v, jnp.int32)
        skey = jnp.where(bits < 0, bits ^ jnp.int32(0x7FFFFFFF), bits)

        def body(j, prefix):
            cand = prefix + (jnp.int32(1) << jnp.int32(31 - j))
            cnt = jnp.sum((skey >= cand).astype(jnp.int32))
            return jnp.where(cnt >= keep, cand, prefix)

        kth = jax.lax.fori_loop(0, 32, body, jnp.int32(-2147483648))
        tau_bits = jnp.where(kth < 0, kth ^ jnp.int32(0x7FFFFFFF), kth)
        tau = jax.lax.bitcast_convert_type(tau_bits, jnp.float32)
        gt = skey > kth
        cnt_gt = jnp.sum(gt.astype(jnp.int32))
        sum_gt = jnp.sum(jnp.where(gt, v, 0.0))
        total = sum_gt + (keep - cnt_gt).astype(jnp.float32) * tau
        o_ref[...] = jnp.reshape(total / keep, (1, 1))


def kernel(input, target):
    B, C = input.shape
    R = 512
    G = B // R
    keep = min(B, int(B * 0.7))
    t3 = target.astype(jnp.int32).reshape(G, 1, R)
    out = pl.pallas_call(
        functools.partial(_ohem_kernel, nblocks=G, keep=keep),
        grid=(G,),
        in_specs=[
            pl.BlockSpec((R, C), lambda i: (i, 0)),
            pl.BlockSpec((1, 1, R), lambda i: (i, 0, 0)),
        ],
        out_specs=pl.BlockSpec((1, 1), lambda i: (0, 0)),
        out_shape=jax.ShapeDtypeStruct((1, 1), jnp.float32),
        scratch_shapes=[pltpu.VMEM((G, R), jnp.float32)],
    )(input, t3)
    return out[0, 0]
